# initial kernel scaffold (unmeasured)
import functools

import jax
import jax.numpy as jnp
from jax import lax
from jax.experimental import pallas as pl
from jax.experimental.pallas import tpu as pltpu

B, H, D = 8, 8, 64
NPAGES_LOCAL = 64
BS = 16
KLOC = NPAGES_LOCAL * BS
NEG = -1e30


def _body(q_ref, k_ref, v_ref, bt_ref, lens_ref, out_ref,
          o_s, stats_s, send_sems, recv_sems):
    my_x = lax.axis_index("x")
    my_y = lax.axis_index("y")
    peer = (1 - my_x, my_y)

    barrier_sem = pltpu.get_barrier_semaphore()
    pl.semaphore_signal(barrier_sem, inc=1, device_id=peer,
                        device_id_type=pl.DeviceIdType.MESH)
    pl.semaphore_wait(barrier_sem, 1)

    bt = bt_ref[:, :]
    lens = lens_ref[:, :]
    j_idx = lax.broadcasted_iota(jnp.int32, (B, 64), 1)
    valid_slot = j_idx < lens
    pids = (lax.broadcasted_iota(jnp.int32, (B, 64, NPAGES_LOCAL), 2)
            + my_x * NPAGES_LOCAL)
    hit = (bt[:, :, None] == pids) & valid_slot[:, :, None]
    cnt = jnp.sum(hit.astype(jnp.float32), axis=1)
    cntk = jnp.broadcast_to(cnt[:, :, None], (B, NPAGES_LOCAL, BS))
    cntk = cntk.reshape(B, KLOC)
    key_valid = cntk > 0.0

    scale = D ** -0.5
    m_cols = []
    l_cols = []
    o_heads = []
    for h in range(H):
        q_h = q_ref[:, h, :]
        k_h = k_ref[:, :, h, :].reshape(KLOC, D)
        v_h = v_ref[:, :, h, :].reshape(KLOC, D)
        s_h = lax.dot_general(
            q_h, k_h, (((1,), (1,)), ((), ())),
            preferred_element_type=jnp.float32) * scale
        s_h = jnp.where(key_valid, s_h, NEG)
        m_h = jnp.max(s_h, axis=1, keepdims=True)
        p_h = jnp.exp(s_h - m_h) * cntk
        l_h = jnp.sum(p_h, axis=1, keepdims=True)
        o_h = lax.dot_general(
            p_h, v_h, (((1,), (0,)), ((), ())),
            preferred_element_type=jnp.float32)
        m_cols.append(m_h)
        l_cols.append(l_h)
        o_heads.append(o_h)

    m_loc = jnp.concatenate(m_cols, axis=1)
    l_loc = jnp.concatenate(l_cols, axis=1)
    o_loc = jnp.stack(o_heads, axis=1)

    o_s[0] = o_loc
    stats_s[0, 0] = m_loc
    stats_s[0, 1] = l_loc

    rdma_o = pltpu.make_async_remote_copy(
        src_ref=o_s.at[0], dst_ref=o_s.at[1],
        send_sem=send_sems.at[0], recv_sem=recv_sems.at[0],
        device_id=peer, device_id_type=pl.DeviceIdType.MESH)
    rdma_s = pltpu.make_async_remote_copy(
        src_ref=stats_s.at[0], dst_ref=stats_s.at[1],
        send_sem=send_sems.at[1], recv_sem=recv_sems.at[1],
        device_id=peer, device_id_type=pl.DeviceIdType.MESH)
    rdma_o.start()
    rdma_s.start()
    rdma_o.wait()
    rdma_s.wait()

    m_rem = stats_s[1, 0]
    l_rem = stats_s[1, 1]
    o_rem = o_s[1]
    m_new = jnp.maximum(m_loc, m_rem)
    a_loc = jnp.exp(m_loc - m_new)
    a_rem = jnp.exp(m_rem - m_new)
    l_tot = l_loc * a_loc + l_rem * a_rem
    o_tot = o_loc * a_loc[:, :, None] + o_rem * a_rem[:, :, None]
    out_ref[:, :, :] = o_tot / l_tot[:, :, None]


def kernel(Q, K, V, bt, lens):
    q2 = Q.reshape(B, H, D)
    lens2 = lens.reshape(B, 1)

    out = pl.pallas_call(
        _body,
        out_shape=jax.ShapeDtypeStruct((B, H, D), jnp.float32),
        in_specs=[
            pl.BlockSpec(memory_space=pltpu.VMEM),
            pl.BlockSpec(memory_space=pltpu.VMEM),
            pl.BlockSpec(memory_space=pltpu.VMEM),
            pl.BlockSpec(memory_space=pltpu.VMEM),
            pl.BlockSpec(memory_space=pltpu.VMEM),
        ],
        out_specs=pl.BlockSpec(memory_space=pltpu.VMEM),
        scratch_shapes=[
            pltpu.VMEM((2, B, H, D), jnp.float32),
            pltpu.VMEM((2, 2, B, H), jnp.float32),
            pltpu.SemaphoreType.DMA((2,)),
            pltpu.SemaphoreType.DMA((2,)),
        ],
        compiler_params=pltpu.CompilerParams(collective_id=0),
    )(q2, K, V, bt, lens2)
    return out.reshape(B, 1, H, D)


# baseline (device time: 16800 ns/iter reference)
import jax
import jax.numpy as jnp
from jax import lax
from jax.experimental import pallas as pl
from jax.experimental.pallas import tpu as pltpu

B, H, D = 8, 8, 64
NPAGES = 128
NPAGES_LOCAL = 64
BS = 16
KLOC = NPAGES_LOCAL * BS
NEG = -1e30


def _body(q_ref, k_ref, v_ref, cntk_ref, out_ref,
          o_s, stats_s, send_sems, recv_sems):
    my_x = lax.axis_index("x")
    my_y = lax.axis_index("y")
    peer = (1 - my_x, my_y)

    barrier_sem = pltpu.get_barrier_semaphore()
    pl.semaphore_signal(barrier_sem, inc=1, device_id=peer,
                        device_id_type=pl.DeviceIdType.MESH)
    pl.semaphore_wait(barrier_sem, 1)

    cntk = cntk_ref[my_x]
    key_valid = cntk > 0.0

    scale = D ** -0.5
    for h in range(H):
        q_h = q_ref[:, h, :]
        k_h = k_ref[:, :, h, :].reshape(KLOC, D)
        v_h = v_ref[:, :, h, :].reshape(KLOC, D)
        s_h = lax.dot_general(
            q_h, k_h, (((1,), (1,)), ((), ())),
            preferred_element_type=jnp.float32) * scale
        s_h = jnp.where(key_valid, s_h, NEG)
        m_h = jnp.max(s_h, axis=1, keepdims=True)
        p_h = jnp.exp(s_h - m_h) * cntk
        l_h = jnp.sum(p_h, axis=1, keepdims=True)
        o_h = lax.dot_general(
            p_h, v_h, (((1,), (0,)), ((), ())),
            preferred_element_type=jnp.float32)
        o_s[0, h] = o_h
        stats_s[0, h, :, 0:1] = m_h
        stats_s[0, h, :, 1:2] = l_h

    rdma_o = pltpu.make_async_remote_copy(
        src_ref=o_s.at[0], dst_ref=o_s.at[1],
        send_sem=send_sems.at[0], recv_sem=recv_sems.at[0],
        device_id=peer, device_id_type=pl.DeviceIdType.MESH)
    rdma_s = pltpu.make_async_remote_copy(
        src_ref=stats_s.at[0], dst_ref=stats_s.at[1],
        send_sem=send_sems.at[1], recv_sem=recv_sems.at[1],
        device_id=peer, device_id_type=pl.DeviceIdType.MESH)
    rdma_o.start()
    rdma_s.start()
    rdma_o.wait()
    rdma_s.wait()

    for h in range(H):
        m_a = stats_s[0, h, :, 0:1]
        l_a = stats_s[0, h, :, 1:2]
        m_b = stats_s[1, h, :, 0:1]
        l_b = stats_s[1, h, :, 1:2]
        m_n = jnp.maximum(m_a, m_b)
        a_sc = jnp.exp(m_a - m_n)
        b_sc = jnp.exp(m_b - m_n)
        l_t = l_a * a_sc + l_b * b_sc
        o_t = o_s[0, h] * a_sc + o_s[1, h] * b_sc
        out_ref[:, h, :] = o_t / l_t


def kernel(Q, K, V, bt, lens):
    q2 = Q.reshape(B, H, D)

    valid = jnp.arange(bt.shape[1])[None, :] < lens[:, None]
    btm = jnp.where(valid, bt, -1)
    pid = jnp.arange(NPAGES, dtype=bt.dtype)
    cnt = jnp.sum(
        (btm[:, :, None] == pid[None, None, :]).astype(jnp.float32), axis=1
    )
    cntk = jnp.broadcast_to(cnt[:, :, None], (B, NPAGES, BS))
    cntk = cntk.reshape(B, 2, KLOC).transpose(1, 0, 2)

    out = pl.pallas_call(
        _body,
        out_shape=jax.ShapeDtypeStruct((B, H, D), jnp.float32),
        in_specs=[
            pl.BlockSpec(memory_space=pltpu.VMEM),
            pl.BlockSpec(memory_space=pltpu.VMEM),
            pl.BlockSpec(memory_space=pltpu.VMEM),
            pl.BlockSpec(memory_space=pltpu.VMEM),
        ],
        out_specs=pl.BlockSpec(memory_space=pltpu.VMEM),
        scratch_shapes=[
            pltpu.VMEM((2, H, B, D), jnp.float32),
            pltpu.VMEM((2, H, B, 2), jnp.float32),
            pltpu.SemaphoreType.DMA((2,)),
            pltpu.SemaphoreType.DMA((2,)),
        ],
        compiler_params=pltpu.CompilerParams(collective_id=0),
    )(q2, K, V, cntk)
    return out.reshape(B, 1, H, D)


# device time: 12765 ns/iter; 1.3161x vs baseline; 1.3161x over previous
import jax
import jax.numpy as jnp
from jax import lax
from jax.experimental import pallas as pl
from jax.experimental.pallas import tpu as pltpu

B, H, D = 8, 8, 64
HY = H // 2
NPAGES = 128
NPAGES_LOCAL = 64
BS = 16
KLOC = NPAGES_LOCAL * BS
NEG = -1e30
_NO_COMM = False


def _body(q_ref, k_hbm, v_hbm, bt_hbm, lens_hbm, out_hbm,
          k_scr, v_scr, bt_scr, lens_scr, out_scr, ol_s,
          copy_sems, send_sem, recv_sem):
    my_x = lax.axis_index("x")
    my_y = lax.axis_index("y")
    x_peer = (1 - my_x, my_y)
    y_peer = (my_x, 1 - my_y)
    d_peer = (1 - my_x, 1 - my_y)
    my_h0 = my_y * HY

    def mine(h):
        return (h >= my_h0) & (h < my_h0 + HY)

    halves = [pl.ds(0, NPAGES_LOCAL // 2),
              pl.ds(NPAGES_LOCAL // 2, NPAGES_LOCAL // 2)]
    kv_dmas = []
    for i, hv in enumerate(halves):
        kv_dmas.append(pltpu.make_async_copy(
            k_hbm.at[hv], k_scr.at[hv], copy_sems.at[i]))
        kv_dmas.append(pltpu.make_async_copy(
            v_hbm.at[hv], v_scr.at[hv], copy_sems.at[2 + i]))
    cb = pltpu.make_async_copy(bt_hbm, bt_scr, copy_sems.at[4])
    cl = pltpu.make_async_copy(lens_hbm, lens_scr, copy_sems.at[5])
    for dma in kv_dmas:
        dma.start()
    cb.start()
    cl.start()

    if not _NO_COMM:
        barrier_sem = pltpu.get_barrier_semaphore()
        for nbr in (x_peer, y_peer, d_peer):
            pl.semaphore_signal(barrier_sem, inc=1, device_id=nbr,
                                device_id_type=pl.DeviceIdType.MESH)
        pl.semaphore_wait(barrier_sem, 3)

    cb.wait()
    cl.wait()
    pid_col = (lax.broadcasted_iota(jnp.int32, (NPAGES_LOCAL, 1), 0)
               + my_x * NPAGES_LOCAL)
    j_row = lax.broadcasted_iota(jnp.int32, (1, 64), 1)
    cnt_cols = []
    for b in range(B):
        row = bt_scr[b:b + 1, :]
        row = jnp.where(j_row < lens_scr[b], row, -1)
        oh = (pid_col == row).astype(jnp.float32)
        cnt_cols.append(jnp.sum(oh, axis=1, keepdims=True))
    cntT = jnp.concatenate(cnt_cols, axis=1)
    expand = (lax.broadcasted_iota(jnp.int32, (NPAGES_LOCAL, KLOC), 0)
              == lax.broadcasted_iota(jnp.int32, (NPAGES_LOCAL, KLOC), 1)
              // BS).astype(jnp.float32)
    cntk = lax.dot_general(cntT, expand, (((0,), (0,)), ((), ())),
                           preferred_element_type=jnp.float32)
    key_valid = cntk > 0.0

    for dma in kv_dmas:
        dma.wait()

    scale = D ** -0.5
    for h in range(H):
        @pl.when(mine(h))
        def _():
            q_h = q_ref[:, h, :]
            k_h = k_scr[:, :, h, :].reshape(KLOC, D)
            v_h = v_scr[:, :, h, :].reshape(KLOC, D)
            s_h = lax.dot_general(
                q_h, k_h, (((1,), (1,)), ((), ())),
                preferred_element_type=jnp.float32) * scale
            s_h = jnp.where(key_valid, s_h, NEG)
            p_h = jnp.exp(s_h) * cntk
            l_h = jnp.sum(p_h, axis=1, keepdims=True)
            o_h = lax.dot_general(
                p_h, v_h, (((1,), (0,)), ((), ())),
                preferred_element_type=jnp.float32)
            ol_s[my_x, h, :, 0:D] = o_h
            ol_s[my_x, h, :, D:D + 1] = l_h

    if not _NO_COMM:
        rdmas = []
        for i, nbr in enumerate((x_peer, y_peer, d_peer)):
            rdma = pltpu.make_async_remote_copy(
                src_ref=ol_s.at[my_x, pl.ds(my_h0, HY)],
                dst_ref=ol_s.at[my_x, pl.ds(my_h0, HY)],
                send_sem=send_sem.at[i], recv_sem=recv_sem.at[i],
                device_id=nbr, device_id_type=pl.DeviceIdType.MESH)
            rdma.start()
            rdmas.append(rdma)
        for rdma in rdmas:
            rdma.wait()

    for h in range(H):
        if _NO_COMM:
            o_t = ol_s[my_x, h, :, 0:D]
            l_t = ol_s[my_x, h, :, D:D + 1]
        else:
            o_t = ol_s[0, h, :, 0:D] + ol_s[1, h, :, 0:D]
            l_t = ol_s[0, h, :, D:D + 1] + ol_s[1, h, :, D:D + 1]
        out_scr[:, 0, h, :] = o_t / l_t
    co = pltpu.make_async_copy(out_scr, out_hbm, copy_sems.at[6])
    co.start()
    co.wait()


def kernel(Q, K, V, bt, lens):
    q2 = Q.reshape(B, H, D)
    k_hbm = pltpu.with_memory_space_constraint(K, pltpu.MemorySpace.HBM)
    v_hbm = pltpu.with_memory_space_constraint(V, pltpu.MemorySpace.HBM)
    bt_hbm = pltpu.with_memory_space_constraint(bt, pltpu.MemorySpace.HBM)
    lens_hbm = pltpu.with_memory_space_constraint(lens, pltpu.MemorySpace.HBM)

    return pl.pallas_call(
        _body,
        out_shape=jax.ShapeDtypeStruct((B, 1, H, D), jnp.float32),
        in_specs=[
            pl.BlockSpec(memory_space=pltpu.VMEM),
            pl.BlockSpec(memory_space=pltpu.MemorySpace.HBM),
            pl.BlockSpec(memory_space=pltpu.MemorySpace.HBM),
            pl.BlockSpec(memory_space=pltpu.MemorySpace.HBM),
            pl.BlockSpec(memory_space=pltpu.MemorySpace.HBM),
        ],
        out_specs=pl.BlockSpec(memory_space=pltpu.MemorySpace.HBM),
        scratch_shapes=[
            pltpu.VMEM((NPAGES_LOCAL, BS, H, D), jnp.float32),
            pltpu.VMEM((NPAGES_LOCAL, BS, H, D), jnp.float32),
            pltpu.VMEM((B, 64), jnp.int32),
            pltpu.SMEM((B,), jnp.int32),
            pltpu.VMEM((B, 1, H, D), jnp.float32),
            pltpu.VMEM((2, H, B, 2 * D), jnp.float32),
            pltpu.SemaphoreType.DMA((7,)),
            pltpu.SemaphoreType.DMA((3,)),
            pltpu.SemaphoreType.DMA((3,)),
        ],
        compiler_params=(None if _NO_COMM
                         else pltpu.CompilerParams(collective_id=0)),
    )(q2, k_hbm, v_hbm, bt_hbm, lens_hbm)
